# Initial kernel scaffold; baseline (speedup 1.0000x reference)
import jax
import jax.numpy as jnp
from jax.experimental import pallas as pl


def kernel(x, edge_index, W, att_src, att_dst, bias, lin_W, lin_b):
    raise NotImplementedError("write your pallas kernel here")


# speedup vs baseline: 16.6113x; 16.6113x over previous
import jax
import jax.numpy as jnp
from jax import lax
from jax.experimental import pallas as pl
from jax.experimental.pallas import tpu as pltpu
from jax.experimental.pallas import tpu_sc as plsc

N = 10000
D_IN = 128
D_OUT = 128
HEADS = 4
RW = 144
NP = 10240
NW = 32
B = 64
NBLK = 162
NPAIR = NBLK // 2
NT = 10016
NPA = 10016
LANES = 16


def _prep_body(x_ref, w_ref, asrc_ref, adst_ref, t_ref, as_ref, ad_ref):
    i = pl.program_id(0)
    bn = x_ref.shape[0]
    h = jnp.dot(x_ref[...], w_ref[...], preferred_element_type=jnp.float32)
    row_ids = i * bn + lax.broadcasted_iota(jnp.int32, (bn, 1), 0)
    ones_col = jnp.where(row_ids < N, 1.0, 0.0).astype(jnp.float32)
    zpad = jnp.zeros((bn, RW - D_OUT - 1), jnp.float32)
    t_parts = []
    as_parts = []
    ad_parts = []
    for k in range(HEADS):
        hk = h[:, k * D_OUT:(k + 1) * D_OUT]
        ak_s = asrc_ref[k:k + 1, :]
        ak_d = adst_ref[k:k + 1, :]
        as_k = jnp.sum(hk * ak_s, axis=1).reshape(1, bn)
        ad_k = jnp.sum(hk * ak_d, axis=1).reshape(1, bn)
        as_parts.append(as_k)
        ad_parts.append(ad_k)
        t_parts.append(
            jnp.concatenate([hk, ones_col, zpad], axis=1).reshape(1, bn, RW))
    t_ref[...] = jnp.concatenate(t_parts, axis=0)
    as_ref[...] = jnp.concatenate(as_parts, axis=0)
    ad_ref[...] = jnp.concatenate(ad_parts, axis=0)


def _prep(x_pad, W, att_src, att_dst):
    bn = 512
    grid = NP // bn
    return pl.pallas_call(
        _prep_body,
        grid=(grid,),
        in_specs=[
            pl.BlockSpec((bn, D_IN), lambda i: (i, 0)),
            pl.BlockSpec((D_IN, HEADS * D_OUT), lambda i: (0, 0)),
            pl.BlockSpec((HEADS, D_OUT), lambda i: (0, 0)),
            pl.BlockSpec((HEADS, D_OUT), lambda i: (0, 0)),
        ],
        out_specs=[
            pl.BlockSpec((HEADS, bn, RW), lambda i: (0, i, 0)),
            pl.BlockSpec((HEADS, bn), lambda i: (0, i)),
            pl.BlockSpec((HEADS, bn), lambda i: (0, i)),
        ],
        out_shape=[
            jax.ShapeDtypeStruct((HEADS, NP, RW), jnp.float32),
            jax.ShapeDtypeStruct((HEADS, NP), jnp.float32),
            jax.ShapeDtypeStruct((HEADS, NP), jnp.float32),
        ],
    )(x_pad, W, att_src, att_dst)


def _leaky(t):
    return jnp.where(t >= 0.0, t, 0.2 * t)


def _zero_rows(rows3):
    def _zb(r, _):
        for b in range(2):
            for c in range(RW // LANES):
                rows3[b, r, pl.ds(c * LANES, LANES)] = jnp.zeros(
                    (LANES,), jnp.float32)
        return 0
    lax.fori_loop(0, B, _zb, 0)


def _zero_acc_slice(rows3, acc, base):
    nfull = (NPA // LANES) // B
    rem = (NPA // LANES) - nfull * B
    for t in range(nfull):
        pltpu.sync_copy(rows3.at[0], acc.at[pl.ds(base + t * B, B)])
    pltpu.sync_copy(rows3.at[0].at[pl.ds(0, rem)],
                    acc.at[pl.ds(base + nfull * B, rem)])


def _scale_rows(rows2, alpha_t):
    @plsc.parallel_loop(0, B, 1, unroll=2)
    def _scale(e):
        av = plsc.load_gather(alpha_t, [jnp.full((LANES,), e, jnp.int32)])
        for c in range(RW // LANES):
            rv = rows2[e, pl.ds(c * LANES, LANES)]
            rows2[e, pl.ds(c * LANES, LANES)] = rv * av


def _sc_body(t_hbm, as_hbm, ad_hbm, src_hbm, dst_hbm, p_hbm,
             as_t, ad_t, src_st, dst_st3, alpha_t, rows3, acc,
             gsem0, gsem1, ssem0, ssem1):
    cid = lax.axis_index("c")
    sid = lax.axis_index("s")
    wid = sid * 2 + cid
    zrows = NPA // LANES
    base = sid * zrows

    _zero_rows(rows3)
    _zero_acc_slice(rows3, acc, base)
    plsc.subcore_barrier()

    for k in range(HEADS):
        pltpu.sync_copy(as_hbm.at[k].at[pl.ds(0, NT)], as_t)
        pltpu.sync_copy(ad_hbm.at[k].at[pl.ds(0, NT)], ad_t)

        def _mx(i, m):
            return jnp.maximum(m, as_t[pl.ds(i * LANES, LANES)])
        m_run = lax.fori_loop(0, NT // LANES, _mx,
                              jnp.full((LANES,), -3.0e38, jnp.float32))
        lanes = lax.broadcasted_iota(jnp.int32, (LANES,), 0)
        mk = m_run
        for s in (8, 4, 2, 1):
            mk = jnp.maximum(mk, mk[lanes ^ s])

        def _alpha_blk(stref_s, stref_d, jj, mk):
            for u in range(B // LANES):
                sv = stref_s[jj, pl.ds(u * LANES, LANES)]
                dv = stref_d[jj, pl.ds(u * LANES, LANES)]
                asv = plsc.load_gather(as_t, [sv])
                adv = plsc.load_gather(ad_t, [dv])
                t = _leaky(asv + adv)
                mv = _leaky(mk + adv)
                alpha_t[pl.ds(u * LANES, LANES)] = jnp.exp(t - mv)

        def _pair(i, _):
            q = lax.rem(i, 2)
            dstq = dst_st3.at[q]
            pltpu.sync_copy(src_hbm.at[wid].at[pl.ds(i * 2, 2)], src_st)
            pltpu.sync_copy(dst_hbm.at[wid].at[pl.ds(i * 2, 2)], dstq)
            g0 = pltpu.async_copy(t_hbm.at[k].at[src_st.at[0]],
                                  rows3.at[0], gsem0)
            _alpha_blk(src_st, dstq, 0, mk)
            g0.wait()
            g1 = pltpu.async_copy(t_hbm.at[k].at[src_st.at[1]],
                                  rows3.at[1], gsem1)
            _scale_rows(rows3.at[0], alpha_t)
            s0 = pltpu.async_copy(rows3.at[0], acc.at[dstq.at[0]],
                                  ssem0, add=True)
            _alpha_blk(src_st, dstq, 1, mk)
            g1.wait()
            _scale_rows(rows3.at[1], alpha_t)
            s1 = pltpu.async_copy(rows3.at[1], acc.at[dstq.at[1]],
                                  ssem1, add=True)
            s0.wait()
            s1.wait()
            return 0

        lax.fori_loop(0, NPAIR, _pair, 0)
        plsc.subcore_barrier()

        pltpu.sync_copy(acc.at[pl.ds(base, zrows)],
                        p_hbm.at[cid, k].at[pl.ds(base, zrows)])
        if k < HEADS - 1:
            _zero_rows(rows3)
            _zero_acc_slice(rows3, acc, base)
            plsc.subcore_barrier()


def _sc_pass(T, AS, AD, SRCr, DSTr):
    mesh = plsc.VectorSubcoreMesh(core_axis_name="c", subcore_axis_name="s")
    return pl.kernel(
        _sc_body,
        out_type=jax.ShapeDtypeStruct((2, HEADS, NPA, RW), jnp.float32),
        mesh=mesh,
        compiler_params=pltpu.CompilerParams(
            needs_layout_passes=False, use_tc_tiling_on_sc=False),
        scratch_types=[
            pltpu.VMEM((NT,), jnp.float32),
            pltpu.VMEM((NT,), jnp.float32),
            pltpu.VMEM((2, B), jnp.int32),
            pltpu.VMEM((2, 2, B), jnp.int32),
            pltpu.VMEM((B,), jnp.float32),
            pltpu.VMEM((2, B, RW), jnp.float32),
            pltpu.VMEM_SHARED((NPA, RW), jnp.float32),
            pltpu.SemaphoreType.DMA,
            pltpu.SemaphoreType.DMA,
            pltpu.SemaphoreType.DMA,
            pltpu.SemaphoreType.DMA,
        ],
    )(T, AS, AD, SRCr, DSTr)


def _post_body(p_ref, bias_ref, lwt_ref, lb_ref, o_ref):
    pp = p_ref[0] + p_ref[1]
    parts = []
    for k in range(HEADS):
        num = pp[k, :, 0:D_OUT]
        den = pp[k, :, D_OUT:D_OUT + 1]
        parts.append(num / (den + 1e-16))
    g = jnp.concatenate(parts, axis=1) + bias_ref[...]
    y = jnp.dot(g, lwt_ref[...], preferred_element_type=jnp.float32)
    y = y + lb_ref[...]
    o_ref[...] = jnp.where(y > 0.0, y, jnp.exp(jnp.minimum(y, 0.0)) - 1.0)


def _post(P, bias, lin_WT, lin_b):
    bn = 400
    grid = N // bn
    return pl.pallas_call(
        _post_body,
        grid=(grid,),
        in_specs=[
            pl.BlockSpec((2, HEADS, bn, RW), lambda i: (0, 0, i, 0)),

            pl.BlockSpec((1, HEADS * D_OUT), lambda i: (0, 0)),
            pl.BlockSpec((HEADS * D_OUT, D_OUT), lambda i: (0, 0)),
            pl.BlockSpec((1, D_OUT), lambda i: (0, 0)),
        ],
        out_specs=pl.BlockSpec((bn, D_OUT), lambda i: (i, 0)),
        out_shape=jax.ShapeDtypeStruct((N, D_OUT), jnp.float32),
    )(P, bias, lin_WT, lin_b)


def kernel(x, edge_index, W, att_src, att_dst, bias, lin_W, lin_b):
    n = x.shape[0]
    e = edge_index.shape[1]
    ep = e + n

    x_pad = jnp.pad(x, ((0, NP - n), (0, 0)))
    att_s = att_src.reshape(HEADS, D_OUT)
    att_d = att_dst.reshape(HEADS, D_OUT)

    loop = jnp.arange(n, dtype=jnp.int32)
    src = jnp.concatenate([edge_index[0].astype(jnp.int32), loop])
    dst = jnp.concatenate([edge_index[1].astype(jnp.int32), loop])
    nblk = NBLK
    ep_pad = NW * nblk * B
    src = jnp.pad(src, (0, ep_pad - ep), constant_values=n)
    dst = jnp.pad(dst, (0, ep_pad - ep), constant_values=n)
    SRCr = src.reshape(NW, nblk, B)
    DSTr = dst.reshape(NW, nblk, B)

    T, AS, AD = _prep(x_pad, W, att_s, att_d)
    P = _sc_pass(T, AS, AD, SRCr, DSTr)
    out = _post(P, bias.reshape(1, HEADS * D_OUT), lin_W.T,
                lin_b.reshape(1, D_OUT))
    return out
